# baseline (device time: 122382 ns/iter reference)
import jax
import jax.numpy as jnp
from jax import lax
from jax.experimental import pallas as pl
from jax.experimental.pallas import tpu as pltpu


def kernel(x):
    M, N = x.shape
    HALF = M // 2

    SIZES = [32, 32, 64, 128] + [256] * 14 + [128, 64, 32, 32]
    assert sum(SIZES) == HALF
    OFFS = [sum(SIZES[:i]) for i in range(len(SIZES))]
    C = len(SIZES)

    def body(x_hbm, out_ref, xf32, ysend, yrecv,
             load_sems, ysend_sems, yrecv_sems, xsend_sems, xrecv_sems):
        my_x = lax.axis_index("x")
        my_y = lax.axis_index("y")
        y_nbr = (my_x, 1 - my_y)
        x_nbr = (1 - my_x, my_y)

        row0 = my_x * HALF
        other_row0 = (1 - my_x) * HALF

        loads = []
        for c in range(C):
            ld = pltpu.make_async_copy(
                x_hbm.at[pl.ds(row0 + OFFS[c], SIZES[c])],
                xf32.at[pl.ds(OFFS[c], SIZES[c])],
                load_sems.at[c],
            )
            ld.start()
            loads.append(ld)

        barrier_sem = pltpu.get_barrier_semaphore()
        for nbr in (y_nbr, x_nbr):
            pl.semaphore_signal(
                barrier_sem, inc=1,
                device_id=nbr, device_id_type=pl.DeviceIdType.MESH,
            )
        pl.semaphore_wait(barrier_sem, 2)

        y_rdmas = []
        for c in range(C):
            sl = pl.ds(OFFS[c], SIZES[c])
            loads[c].wait()
            ysend[sl, :] = xf32[sl, :].astype(jnp.bfloat16)
            r = pltpu.make_async_remote_copy(
                src_ref=ysend.at[sl],
                dst_ref=yrecv.at[sl],
                send_sem=ysend_sems.at[c],
                recv_sem=yrecv_sems.at[c],
                device_id=y_nbr,
                device_id_type=pl.DeviceIdType.MESH,
            )
            r.start()
            y_rdmas.append(r)

        x_rdmas = []
        for c in range(C):
            sl = pl.ds(OFFS[c], SIZES[c])
            out_sl = pl.ds(row0 + OFFS[c], SIZES[c])
            y_rdmas[c].wait_recv()
            out_ref[out_sl, :] = ysend[sl, :] + yrecv[sl, :]
            r = pltpu.make_async_remote_copy(
                src_ref=out_ref.at[out_sl],
                dst_ref=out_ref.at[out_sl],
                send_sem=xsend_sems.at[c],
                recv_sem=xrecv_sems.at[c],
                device_id=x_nbr,
                device_id_type=pl.DeviceIdType.MESH,
            )
            r.start()
            x_rdmas.append(r)

        for c in range(C):
            recv = pltpu.make_async_remote_copy(
                src_ref=out_ref.at[pl.ds(row0, SIZES[c])],
                dst_ref=out_ref.at[pl.ds(other_row0 + OFFS[c], SIZES[c])],
                send_sem=xsend_sems.at[c],
                recv_sem=xrecv_sems.at[c],
                device_id=x_nbr,
                device_id_type=pl.DeviceIdType.MESH,
            )
            recv.wait_recv()
        for c in range(C):
            y_rdmas[c].wait_send()
            x_rdmas[c].wait_send()

    return pl.pallas_call(
        body,
        out_shape=jax.ShapeDtypeStruct((M, N), jnp.bfloat16),
        in_specs=[pl.BlockSpec(memory_space=pl.ANY)],
        out_specs=pl.BlockSpec(memory_space=pltpu.MemorySpace.VMEM),
        scratch_shapes=[
            pltpu.VMEM((HALF, N), jnp.float32),
            pltpu.VMEM((HALF, N), jnp.bfloat16),
            pltpu.VMEM((HALF, N), jnp.bfloat16),
            pltpu.SemaphoreType.DMA((C,)),
            pltpu.SemaphoreType.DMA((C,)),
            pltpu.SemaphoreType.DMA((C,)),
            pltpu.SemaphoreType.DMA((C,)),
            pltpu.SemaphoreType.DMA((C,)),
        ],
        compiler_params=pltpu.CompilerParams(
            collective_id=0,
            vmem_limit_bytes=100 * 1024 * 1024,
        ),
    )(x)
